# Initial kernel scaffold; baseline (speedup 1.0000x reference)
import jax
import jax.numpy as jnp
from jax.experimental import pallas as pl


def kernel(species, positions):
    raise NotImplementedError("write your pallas kernel here")


# speedup vs baseline: 123.0162x; 123.0162x over previous
import math

import numpy as np
import jax
import jax.numpy as jnp
from jax.experimental import pallas as pl
from jax.experimental.pallas import tpu as pltpu

_N = 160
_RCR = 5.1
_RCA = 3.5
_ETA_R = 19.7
_ETA_A = 12.5
_ZETA = 14.1
_SHF_R = (0.8, 1.06875, 1.3375, 1.60625, 1.875, 2.14375, 2.4125, 2.68125,
          2.95, 3.21875, 3.4875, 3.75625, 4.025, 4.29375, 4.5625, 4.83125)
_SHF_Z = (0.19634954, 0.58904862, 0.9817477, 1.3744468,
          1.7671459, 2.1598449, 2.552544, 2.9452431)
_SHF_A = (0.8, 1.1375, 1.475, 1.8125, 2.15, 2.4875, 2.825, 3.1625)
_NCOLS = 7 * 16 + 28 * 8 * 8
_B = 8
_STEPS = _N // _B
_PI = math.pi
_SCALE = 1.0 / (_N * _NCOLS)


def _aev_kernel(post_ref, posc_ref, out_ref):
    step = pl.program_id(0)
    base = step * _B
    f32 = jnp.float32

    px = post_ref[0:1, :]
    py = post_ref[1:2, :]
    pz = post_ref[2:3, :]
    cblk = posc_ref[pl.ds(base, _B), :]
    cx = cblk[:, 0:1]
    cy = cblk[:, 1:2]
    cz = cblk[:, 2:3]

    dx = px - cx
    dy = py - cy
    dz = pz - cz
    d2 = dx * dx + dy * dy + dz * dz
    valid = d2 > 1e-12
    dij = jnp.where(valid, jnp.sqrt(jnp.where(valid, d2, 1.0)), 0.0)

    jidx = jax.lax.broadcasted_iota(jnp.int32, (_B, _N), 1)
    cidx = jax.lax.broadcasted_iota(jnp.int32, (_B, _N), 0) + base
    ne = jidx != cidx

    in_r = ((dij <= _RCR) & ne).astype(f32)
    fc_r = jnp.where(dij <= _RCR, 0.5 * jnp.cos(_PI / _RCR * dij) + 0.5, 0.0) * in_r
    racc = jnp.zeros((_B, _N), f32)
    for s in _SHF_R:
        racc += jnp.exp(-_ETA_R * (dij - s) ** 2)
    radial_part = jnp.sum(0.25 * racc * fc_r)

    in_a = ((dij <= _RCA) & ne).astype(f32)
    fcj = jnp.where(dij <= _RCA, 0.5 * jnp.cos(_PI / _RCA * dij) + 0.5, 0.0) * in_a

    dots = (dx[:, :, None] * dx[:, None, :]
            + dy[:, :, None] * dy[:, None, :]
            + dz[:, :, None] * dz[:, None, :])
    denom = jnp.maximum(dij[:, :, None] * dij[:, None, :], 1e-10)
    xang = 0.95 * dots / denom
    yang = jnp.sqrt(jnp.maximum(1.0 - xang * xang, 0.0))

    f1 = jnp.zeros((_B, _N, _N), f32)
    for s in _SHF_Z:
        t = 0.5 + 0.5 * (xang * np.float32(np.cos(s)) + yang * np.float32(np.sin(s)))
        f1 += jnp.exp(_ZETA * jnp.log(jnp.maximum(t, 1e-6)))

    avg = 0.5 * (dij[:, :, None] + dij[:, None, :])
    f2 = jnp.zeros((_B, _N, _N), f32)
    for s in _SHF_A:
        f2 += jnp.exp(-_ETA_A * (avg - s) ** 2)

    fcp = fcj[:, :, None] * fcj[:, None, :]
    jj = jax.lax.broadcasted_iota(jnp.int32, (_B, _N, _N), 1)
    kk = jax.lax.broadcasted_iota(jnp.int32, (_B, _N, _N), 2)
    kmask = (jj != kk).astype(f32)
    angular_part = jnp.sum(f1 * f2 * fcp * kmask)

    @pl.when(step == 0)
    def _init():
        out_ref[:, :] = jnp.zeros((1, 1), f32)

    out_ref[:, :] += (radial_part + angular_part) * _SCALE


def kernel(species, positions):
    del species
    post = positions.T.astype(jnp.float32)
    out = pl.pallas_call(
        _aev_kernel,
        grid=(_STEPS,),
        in_specs=[
            pl.BlockSpec((3, _N), lambda i: (0, 0)),
            pl.BlockSpec((_N, 3), lambda i: (0, 0)),
        ],
        out_specs=pl.BlockSpec((1, 1), lambda i: (0, 0)),
        out_shape=jax.ShapeDtypeStruct((1, 1), jnp.float32),
    )(post, positions)
    return out[0, 0]
